# baseline (speedup 1.0000x reference)
import functools
import jax
import jax.numpy as jnp
from jax import lax
from jax.experimental import pallas as pl
from jax.experimental.pallas import tpu as pltpu
from jax.experimental.pallas import tpu_sc as plsc

_NC, _NS = 2, 16
_NW = _NC * _NS
_C = 16
_POS_OFFSET = 2


def _sc_call(xf, table, S):
    R, D = xf.shape
    rpw = R // _NW
    nchunk = rpw // _C
    mesh = plsc.VectorSubcoreMesh(core_axis_name="c", subcore_axis_name="s")

    @functools.partial(
        pl.kernel,
        out_type=jax.ShapeDtypeStruct((R, D), jnp.float32),
        mesh=mesh,
        scratch_types=[
            pltpu.VMEM((2, _C), jnp.int32),
            pltpu.VMEM((2, _C, D), jnp.float32),
            pltpu.VMEM((2, _C, D), jnp.float32),
            pltpu.SemaphoreType.DMA((2,)),
            pltpu.SemaphoreType.DMA((2,)),
            pltpu.SemaphoreType.DMA((2,)),
        ],
    )
    def sc_kernel(x_hbm, t_hbm, o_hbm, idx_v, xv, tv, sx, st, so):
        wid = lax.axis_index("s") * _NC + lax.axis_index("c")
        base = wid * rpw
        s0 = lax.rem(base, S)

        def start_in(ci, slot):
            row0 = base + ci * _C
            idx_v[slot, :] = lax.iota(jnp.int32, _C) + (s0 + ci * _C + _POS_OFFSET)
            pltpu.make_async_copy(
                x_hbm.at[pl.ds(row0, _C), :], xv.at[slot], sx.at[slot]
            ).start()
            pltpu.make_async_copy(
                t_hbm.at[idx_v.at[slot]], tv.at[slot], st.at[slot]
            ).start()

        start_in(0, 0)
        start_in(1, 1)

        @pl.loop(0, nchunk, step=2)
        def _(g):
            for slot in range(2):
                ci = g + slot
                pltpu.make_async_copy(
                    x_hbm.at[pl.ds(0, _C), :], xv.at[slot], sx.at[slot]
                ).wait()
                pltpu.make_async_copy(
                    t_hbm.at[pl.ds(0, _C), :], tv.at[slot], st.at[slot]
                ).wait()

                @pl.loop(0, _C)
                def _(r):
                    for l in range(D // 16):
                        sl = pl.ds(l * 16, 16)
                        xv[slot, r, sl] = xv[slot, r, sl] + tv[slot, r, sl]

                row0 = base + ci * _C
                out_cp = pltpu.make_async_copy(
                    xv.at[slot], o_hbm.at[pl.ds(row0, _C), :], so.at[slot]
                )
                out_cp.start()
                out_cp.wait()

                @pl.when(ci + 2 < nchunk)
                def _():
                    start_in(ci + 2, slot)

    return sc_kernel(xf, table)


@jax.jit
def kernel(x, table):
    B, S, D = x.shape
    out = _sc_call(x.reshape(B * S, D), table, S)
    return out.reshape(B, S, D)


# speedup vs baseline: 1.2247x; 1.2247x over previous
import functools
import jax
import jax.numpy as jnp
from jax import lax
from jax.experimental import pallas as pl
from jax.experimental.pallas import tpu as pltpu
from jax.experimental.pallas import tpu_sc as plsc

_NC, _NS = 2, 16
_NW = _NC * _NS
_C = 16
_POS_OFFSET = 2


def _sc_call(xf, table, S):
    R, D = xf.shape
    rpw = R // _NW
    nchunk = rpw // _C
    mesh = plsc.VectorSubcoreMesh(core_axis_name="c", subcore_axis_name="s")

    @functools.partial(
        pl.kernel,
        out_type=jax.ShapeDtypeStruct((R, D), jnp.float32),
        mesh=mesh,
        scratch_types=[
            pltpu.VMEM((2, _C), jnp.int32),
            pltpu.VMEM((2, _C, D), jnp.float32),
            pltpu.VMEM((2, _C, D), jnp.float32),
            pltpu.SemaphoreType.DMA((2,)),
            pltpu.SemaphoreType.DMA((2,)),
            pltpu.SemaphoreType.DMA((2,)),
        ],
    )
    def sc_kernel(x_hbm, t_hbm, o_hbm, idx_v, xv, tv, sx, st, so):
        wid = lax.axis_index("s") * _NC + lax.axis_index("c")
        base = wid * rpw
        s0 = lax.rem(base, S)

        def start_in(ci, slot):
            row0 = base + ci * _C
            idx_v[slot, :] = lax.iota(jnp.int32, _C) + (s0 + ci * _C + _POS_OFFSET)
            pltpu.make_async_copy(
                x_hbm.at[pl.ds(row0, _C), :], xv.at[slot], sx.at[slot]
            ).start()
            pltpu.make_async_copy(
                t_hbm.at[idx_v.at[slot]], tv.at[slot], st.at[slot]
            ).start()

        start_in(0, 0)
        start_in(1, 1)

        @pl.loop(0, nchunk, step=2)
        def _(g):
            for slot in range(2):
                ci = g + slot
                pltpu.make_async_copy(
                    x_hbm.at[pl.ds(0, _C), :], xv.at[slot], sx.at[slot]
                ).wait()
                pltpu.make_async_copy(
                    t_hbm.at[pl.ds(0, _C), :], tv.at[slot], st.at[slot]
                ).wait()

                @pl.loop(0, _C)
                def _(r):
                    for l in range(D // 16):
                        sl = pl.ds(l * 16, 16)
                        plsc.addupdate(xv.at[slot, r, sl], tv[slot, r, sl])

                row0 = base + ci * _C
                out_cp = pltpu.make_async_copy(
                    xv.at[slot], o_hbm.at[pl.ds(row0, _C), :], so.at[slot]
                )
                out_cp.start()
                out_cp.wait()

                @pl.when(ci + 2 < nchunk)
                def _():
                    start_in(ci + 2, slot)

    return sc_kernel(xf, table)


@jax.jit
def kernel(x, table):
    B, S, D = x.shape
    out = _sc_call(x.reshape(B * S, D), table, S)
    return out.reshape(B, S, D)


# speedup vs baseline: 1.3325x; 1.0880x over previous
import functools
import jax
import jax.numpy as jnp
from jax import lax
from jax.experimental import pallas as pl
from jax.experimental.pallas import tpu as pltpu
from jax.experimental.pallas import tpu_sc as plsc

_NC, _NS = 2, 16
_NW = _NC * _NS
_C = 16
_POS_OFFSET = 2


def _sc_call(xf, table, S):
    R, D = xf.shape
    rpw = R // _NW
    nchunk = rpw // _C
    mesh = plsc.VectorSubcoreMesh(core_axis_name="c", subcore_axis_name="s")

    @functools.partial(
        pl.kernel,
        out_type=jax.ShapeDtypeStruct((R, D), jnp.float32),
        mesh=mesh,
        scratch_types=[
            pltpu.VMEM((2, _C), jnp.int32),
            pltpu.VMEM((4, _C, D), jnp.float32),
            pltpu.VMEM((2, _C, D), jnp.float32),
            pltpu.SemaphoreType.DMA((4,)),
            pltpu.SemaphoreType.DMA((2,)),
            pltpu.SemaphoreType.DMA((4,)),
        ],
    )
    def sc_kernel(x_hbm, t_hbm, o_hbm, idx_v, xv, tv, sx, st, so):
        wid = lax.axis_index("s") * _NC + lax.axis_index("c")
        base = wid * rpw
        s0 = lax.rem(base, S)

        def start_in(ci, xs, ts):
            row0 = base + ci * _C
            idx_v[ts, :] = lax.iota(jnp.int32, _C) + (s0 + ci * _C + _POS_OFFSET)
            pltpu.make_async_copy(
                x_hbm.at[pl.ds(row0, _C), :], xv.at[xs], sx.at[xs]
            ).start()
            pltpu.make_async_copy(
                t_hbm.at[idx_v.at[ts]], tv.at[ts], st.at[ts]
            ).start()

        def wait_out(xs):
            pltpu.make_async_copy(
                xv.at[xs], o_hbm.at[pl.ds(0, _C), :], so.at[xs]
            ).wait()

        start_in(0, 0, 0)
        start_in(1, 1, 1)

        @pl.loop(0, nchunk, step=4)
        def _(g):
            for k in range(4):
                ci = g + k
                xs, ts = k, k % 2
                pltpu.make_async_copy(
                    x_hbm.at[pl.ds(0, _C), :], xv.at[xs], sx.at[xs]
                ).wait()
                pltpu.make_async_copy(
                    t_hbm.at[pl.ds(0, _C), :], tv.at[ts], st.at[ts]
                ).wait()

                @pl.loop(0, _C)
                def _(r):
                    for l in range(D // 16):
                        sl = pl.ds(l * 16, 16)
                        plsc.addupdate(xv.at[xs, r, sl], tv[ts, r, sl])

                row0 = base + ci * _C
                pltpu.make_async_copy(
                    xv.at[xs], o_hbm.at[pl.ds(row0, _C), :], so.at[xs]
                ).start()

                @pl.when(ci >= 2)
                def _():
                    wait_out((k + 2) % 4)

                @pl.when(ci + 2 < nchunk)
                def _():
                    start_in(ci + 2, (k + 2) % 4, ts)

        wait_out((nchunk - 2) % 4)
        wait_out((nchunk - 1) % 4)

    return sc_kernel(xf, table)


@jax.jit
def kernel(x, table):
    B, S, D = x.shape
    out = _sc_call(x.reshape(B * S, D), table, S)
    return out.reshape(B, S, D)


# speedup vs baseline: 1.3677x; 1.0264x over previous
import functools
import jax
import jax.numpy as jnp
from jax import lax
from jax.experimental import pallas as pl
from jax.experimental.pallas import tpu as pltpu
from jax.experimental.pallas import tpu_sc as plsc

_NC, _NS = 2, 16
_NW = _NC * _NS
_C = 16
_POS_OFFSET = 2


def _sc_call(xf, table, S):
    R, D = xf.shape
    rpw = R // _NW
    nchunk = rpw // _C
    mesh = plsc.VectorSubcoreMesh(core_axis_name="c", subcore_axis_name="s")

    @functools.partial(
        pl.kernel,
        out_type=jax.ShapeDtypeStruct((R, D), jnp.float32),
        mesh=mesh,
        scratch_types=[
            pltpu.VMEM((2, _C), jnp.int32),
            pltpu.VMEM((4, _C, D), jnp.float32),
            pltpu.VMEM((2, _C, D), jnp.float32),
            pltpu.SemaphoreType.DMA((4,)),
            pltpu.SemaphoreType.DMA((2,)),
            pltpu.SemaphoreType.DMA((4,)),
        ],
    )
    def sc_kernel(x_hbm, t_hbm, o_hbm, idx_v, xv, tv, sx, st, so):
        wid = lax.axis_index("s") * _NC + lax.axis_index("c")
        base = wid * rpw
        s0 = lax.rem(base, S)

        def start_in(ci, xs, ts):
            row0 = base + ci * _C
            idx_v[ts, :] = lax.iota(jnp.int32, _C) + (s0 + ci * _C + _POS_OFFSET)
            pltpu.make_async_copy(
                x_hbm.at[pl.ds(row0, _C), :], xv.at[xs], sx.at[xs]
            ).start()
            pltpu.make_async_copy(
                t_hbm.at[idx_v.at[ts]], tv.at[ts], st.at[ts]
            ).start()

        def wait_out(xs):
            pltpu.make_async_copy(
                xv.at[xs], o_hbm.at[pl.ds(0, _C), :], so.at[xs]
            ).wait()

        start_in(0, 0, 0)
        start_in(1, 1, 1)

        @pl.loop(0, nchunk, step=4)
        def _(g):
            for k in range(4):
                ci = g + k
                xs, ts = k, k % 2
                pltpu.make_async_copy(
                    x_hbm.at[pl.ds(0, _C), :], xv.at[xs], sx.at[xs]
                ).wait()
                pltpu.make_async_copy(
                    t_hbm.at[pl.ds(0, _C), :], tv.at[ts], st.at[ts]
                ).wait()

                @pl.loop(0, _C, step=2)
                def _(r0):
                    for dr in range(2):
                        for l in range(D // 16):
                            sl = pl.ds(l * 16, 16)
                            plsc.addupdate(xv.at[xs, r0 + dr, sl], tv[ts, r0 + dr, sl])

                row0 = base + ci * _C
                pltpu.make_async_copy(
                    xv.at[xs], o_hbm.at[pl.ds(row0, _C), :], so.at[xs]
                ).start()

                @pl.when(ci >= 2)
                def _():
                    wait_out((k + 2) % 4)

                @pl.when(ci + 2 < nchunk)
                def _():
                    start_in(ci + 2, (k + 2) % 4, ts)

        wait_out((nchunk - 2) % 4)
        wait_out((nchunk - 1) % 4)

    return sc_kernel(xf, table)


@jax.jit
def kernel(x, table):
    B, S, D = x.shape
    out = _sc_call(x.reshape(B * S, D), table, S)
    return out.reshape(B, S, D)


# speedup vs baseline: 2.4805x; 1.8136x over previous
import jax
import jax.numpy as jnp
from jax.experimental import pallas as pl
from jax.experimental.pallas import tpu as pltpu

_BS = 512
_POS_OFFSET = 2


def _tc_body(table_ref, x_ref, o_ref, tbl_v, sems):
    j = pl.program_id(0)
    n = pl.num_programs(0)

    def _copy(k, slot):
        return pltpu.make_async_copy(
            table_ref.at[pl.ds(k * _BS, _BS + 8), :],
            tbl_v.at[slot],
            sems.at[slot],
        )

    @pl.when(j == 0)
    def _():
        _copy(0, 0).start()

    @pl.when(j + 1 < n)
    def _():
        _copy(j + 1, jax.lax.rem(j + 1, 2)).start()

    slot = jax.lax.rem(j, 2)
    _copy(j, slot).wait()
    o_ref[...] = x_ref[...] + tbl_v[slot, _POS_OFFSET:_POS_OFFSET + _BS, :][None, :, :]


@jax.jit
def kernel(x, table):
    B, S, D = x.shape
    n = S // _BS
    return pl.pallas_call(
        _tc_body,
        grid=(n,),
        in_specs=[
            pl.BlockSpec(memory_space=pl.ANY),
            pl.BlockSpec((B, _BS, D), lambda j: (0, j, 0)),
        ],
        out_specs=pl.BlockSpec((B, _BS, D), lambda j: (0, j, 0)),
        out_shape=jax.ShapeDtypeStruct(x.shape, x.dtype),
        scratch_shapes=[
            pltpu.VMEM((2, _BS + 8, D), x.dtype),
            pltpu.SemaphoreType.DMA((2,)),
        ],
    )(table, x)
